# Initial kernel scaffold; baseline (speedup 1.0000x reference)
import jax
import jax.numpy as jnp
from jax.experimental import pallas as pl


def kernel(dr_vec, Z, neighbor_idxs, embeddings):
    raise NotImplementedError("write your pallas kernel here")


# speedup vs baseline: 10.1087x; 10.1087x over previous
import numpy as np
import jax
import jax.numpy as jnp
from jax.experimental import pallas as pl

_N_ATOMS = 10000
_N_EDGES = 320000
_N_RADIAL = 5
_N_BASIS = 7
_R_MIN = 0.5
_R_MAX = 6.0
_BETTA = _N_BASIS ** 2 / _R_MAX ** 2
_RAD_NORM = (2.0 * _BETTA / np.pi) ** 0.25
_EMBED_NORM = 1.0 / np.sqrt(_N_BASIS)
_SHIFTS = (_R_MIN + (_R_MAX - _R_MIN) / _N_BASIS * np.arange(_N_BASIS)).astype(np.float32)

_BE = 6400
_BA = 200

def _om0(r):            return r
def _om1(r, i):         return 5 + r * 3 + i
def _om2(r, i, j):      return 20 + r * 9 + i * 3 + j
def _om3(r, i, j, k):   return 65 + r * 27 + i * 9 + j * 3 + k

_T2 = [(i, j) for i in range(_N_RADIAL) for j in range(i + 1)]
_T3 = [(i, j, k) for i in range(_N_RADIAL) for j in range(i + 1) for k in range(j + 1)]

def _build_matrices():
    r1 = np.zeros((5, 15), np.float32); t1 = np.zeros((3, 15), np.float32)
    for r in range(5):
        for i in range(3):
            r1[r, r * 3 + i] = 1.0; t1[i, r * 3 + i] = 1.0
    r2 = np.zeros((15, 45), np.float32); t2 = np.zeros((3, 45), np.float32)
    for x in range(15):
        for j in range(3):
            r2[x, x * 3 + j] = 1.0; t2[j, x * 3 + j] = 1.0
    r3 = np.zeros((45, 135), np.float32); t3 = np.zeros((3, 135), np.float32)
    for y in range(45):
        for k in range(3):
            r3[y, y * 3 + k] = 1.0; t3[k, y * 3 + k] = 1.0
    btile = np.zeros((7, 35), np.float32); ssum = np.zeros((35, 5), np.float32)
    for r in range(5):
        for b in range(7):
            btile[b, r * 7 + b] = 1.0; ssum[r * 7 + b, r] = 1.0

    pairs = []
    for o, (r, s) in enumerate(_T2):
        for i in range(3):
            pairs.append((_om1(r, i), _om1(s, i), 5 + o))
        for i in range(3):
            for j in range(3):
                pairs.append((_om2(r, i, j), _om2(s, i, j), 20 + o))
        for i in range(3):
            for j in range(3):
                for k in range(3):
                    pairs.append((_om3(r, i, j, k), _om3(s, i, j, k), 35 + o))
    trips = []
    for o, (r, s, t) in enumerate(_T3):
        for i in range(3):
            for j in range(3):
                for k in range(3):
                    trips.append((_om2(r, i, j), _om2(s, i, k), _om2(t, j, k), 50 + o))
    for o, (r, s) in enumerate(_T2):
        for t in range(5):
            out = 85 + o * 5 + t
            for i in range(3):
                for j in range(3):
                    trips.append((_om1(r, i), _om1(s, j), _om2(t, i, j), out))
    for o, (r, s) in enumerate(_T2):
        for t in range(5):
            out = 160 + o * 5 + t
            for i in range(3):
                for j in range(3):
                    for k in range(3):
                        for l in range(3):
                            trips.append((_om3(r, i, j, k), _om3(s, i, j, l), _om2(t, k, l), out))
    for o, (r, s) in enumerate(_T2):
        for t in range(5):
            out = 235 + o * 5 + t
            for i in range(3):
                for j in range(3):
                    for k in range(3):
                        trips.append((_om3(r, i, j, k), _om2(s, i, j), _om1(t, k), out))

    npair, ntrip = len(pairs), len(trips)
    a2 = np.zeros((200, npair), np.float32); b2 = np.zeros((200, npair), np.float32)
    w2 = np.zeros((npair, 310), np.float32)
    for c, (a, b, o) in enumerate(pairs):
        a2[a, c] = 1.0; b2[b, c] = 1.0; w2[c, o] = 1.0
    a3 = np.zeros((200, ntrip), np.float32); b3 = np.zeros((200, ntrip), np.float32)
    c3m = np.zeros((200, ntrip), np.float32); w3 = np.zeros((ntrip, 310), np.float32)
    for c, (a, b, cc, o) in enumerate(trips):
        a3[a, c] = 1.0; b3[b, c] = 1.0; c3m[cc, c] = 1.0; w3[c, o] = 1.0
    w0 = np.zeros((200, 310), np.float32)
    for r in range(5):
        w0[_om0(r), r] = 1.0
    return (r1, t1, r2, t2, r3, t3, btile, ssum), (w0, a2, b2, w2, a3, b3, c3m, w3)

_EDGE_MATS, _ATOM_MATS = _build_matrices()
_EDGE_MATS = tuple(jnp.asarray(m) for m in _EDGE_MATS)
_ATOM_MATS = tuple(jnp.asarray(m) for m in _ATOM_MATS)
_SHIFTS2D = jnp.asarray(_SHIFTS[None, :])


def _edge_kernel(drv_ref, coeff_ref, shifts_ref,
                 r1_ref, t1_ref, r2_ref, t2_ref, r3_ref, t3_ref,
                 btile_ref, ssum_ref, out_ref):
    drv = drv_ref[:, :]
    coeffs = coeff_ref[:, :]
    dr2 = jnp.sum(drv * drv, axis=1, keepdims=True)
    dr = jnp.sqrt(dr2)
    dn = drv / (dr + 1e-5)
    dist = shifts_ref[:, :] - dr
    basis = _RAD_NORM * jnp.exp(-_BETTA * dist * dist)
    basis35 = jnp.dot(basis, btile_ref[:, :])
    rf = jnp.dot(coeffs * basis35, ssum_ref[:, :]) * _EMBED_NORM
    drc = jnp.minimum(dr, _R_MAX)
    cutoff = 0.5 * (jnp.cos(np.pi * drc / _R_MAX) + 1.0)
    rf = rf * cutoff
    first = jnp.dot(rf, r1_ref[:, :]) * jnp.dot(dn, t1_ref[:, :])
    second = jnp.dot(first, r2_ref[:, :]) * jnp.dot(dn, t2_ref[:, :])
    third = jnp.dot(second, r3_ref[:, :]) * jnp.dot(dn, t3_ref[:, :])
    out_ref[:, :] = jnp.concatenate([rf, first, second, third], axis=1)


def _atom_kernel(m_ref, w0_ref, a2_ref, b2_ref, w2_ref,
                 a3_ref, b3_ref, c3_ref, w3_ref, out_ref):
    m = m_ref[:, :]
    out = jnp.dot(m, w0_ref[:, :])
    p2 = jnp.dot(m, a2_ref[:, :]) * jnp.dot(m, b2_ref[:, :])
    out = out + jnp.dot(p2, w2_ref[:, :])
    p3 = jnp.dot(m, a3_ref[:, :]) * jnp.dot(m, b3_ref[:, :]) * jnp.dot(m, c3_ref[:, :])
    out = out + jnp.dot(p3, w3_ref[:, :])
    out_ref[:, :] = out


def _full(shape):
    return pl.BlockSpec(shape, lambda g: (0, 0))


def kernel(dr_vec, Z, neighbor_idxs, embeddings):
    dr_vec = dr_vec.astype(jnp.float32)
    idx_i, idx_j = neighbor_idxs[0], neighbor_idxs[1]
    coeffs = embeddings[Z[idx_j], Z[idx_i]].reshape(_N_EDGES, _N_RADIAL * _N_BASIS)

    r1, t1, r2, t2, r3, t3, btile, ssum = _EDGE_MATS
    edge_m = pl.pallas_call(
        _edge_kernel,
        grid=(_N_EDGES // _BE,),
        in_specs=[
            pl.BlockSpec((_BE, 3), lambda g: (g, 0)),
            pl.BlockSpec((_BE, 35), lambda g: (g, 0)),
            _full((1, 7)),
            _full((5, 15)), _full((3, 15)),
            _full((15, 45)), _full((3, 45)),
            _full((45, 135)), _full((3, 135)),
            _full((7, 35)), _full((35, 5)),
        ],
        out_specs=pl.BlockSpec((_BE, 200), lambda g: (g, 0)),
        out_shape=jax.ShapeDtypeStruct((_N_EDGES, 200), jnp.float32),
    )(dr_vec, coeffs, _SHIFTS2D, r1, t1, r2, t2, r3, t3, btile, ssum)

    moments = jax.ops.segment_sum(edge_m, idx_j, _N_ATOMS)

    w0, a2, b2, w2, a3, b3, c3m, w3 = _ATOM_MATS
    out = pl.pallas_call(
        _atom_kernel,
        grid=(_N_ATOMS // _BA,),
        in_specs=[
            pl.BlockSpec((_BA, 200), lambda g: (g, 0)),
            _full(w0.shape), _full(a2.shape), _full(b2.shape), _full(w2.shape),
            _full(a3.shape), _full(b3.shape), _full(c3m.shape), _full(w3.shape),
        ],
        out_specs=pl.BlockSpec((_BA, 310), lambda g: (g, 0)),
        out_shape=jax.ShapeDtypeStruct((_N_ATOMS, 310), jnp.float32),
    )(moments, w0, a2, b2, w2, a3, b3, c3m, w3)
    return out
